# Initial kernel scaffold; baseline (speedup 1.0000x reference)
import jax
import jax.numpy as jnp
from jax.experimental import pallas as pl


def kernel(gene_value_ng, total_mrna_umis_n, gene_id_g, cell_type_n, tissue_n):
    raise NotImplementedError("write your pallas kernel here")


# speedup vs baseline: 1.6800x; 1.6800x over previous
import functools

import jax
import jax.numpy as jnp
from jax import lax
from jax.experimental import pallas as pl
from jax.experimental.pallas import tpu as pltpu
from jax.experimental.pallas import tpu_sc as plsc

N = 1024
G = 8192
CONTEXT_LEN = 2048
M = 2
C = CONTEXT_LEN - M
CP = 2048
PREFIX_LEN = 1024
MIN_UMIS = 1000.0
MAX_UMIS = 100000.0
GENE_VALUE_VOCAB = 2048
CELL_TYPE_VOCAB = 100
TISSUE_VOCAB = 50

LANES = 16
STEPS = G // LANES
NW = 32
ROWS_PER_W = N // NW


def _sc_sort_gather_body(u_hbm, gv_hbm, gid_hbm, oid_hbm, oval_hbm,
                         u_v, a_v, b_v, c_v, gvrow_v, gid_v, hist_v,
                         oidx_v, oval_v, sem):
    wid = lax.axis_index("s") * 2 + lax.axis_index("c")
    iota = lax.iota(jnp.int32, LANES)
    ones = iota - iota + 1

    pltpu.sync_copy(gid_hbm, gid_v)

    def transpose_pos(q):
        return ((q & (STEPS - 1)) << 4) | (q >> 9)

    def do_pass(digit_ref, payload_ref, out_ref, shift, mask, nbins,
                repack, final):
        def zero_step(bb, _):
            hist_v[pl.ds(bb * LANES, LANES)] = iota - iota
            return 0
        lax.fori_loop(0, nbins, zero_step, 0)

        def hist_step(t, _):
            w = digit_ref[pl.ds(t * LANES, LANES)]
            d = (w >> shift) & mask
            plsc.addupdate_scatter(hist_v, [(d << 4) | iota], ones)
            return 0
        lax.fori_loop(0, STEPS, hist_step, 0)

        def scan_step(bb, carry):
            h = hist_v[pl.ds(bb * LANES, LANES)]
            inc = plsc.cumsum(h)
            hist_v[pl.ds(bb * LANES, LANES)] = inc - h + carry
            return carry + jnp.sum(h)
        lax.fori_loop(0, nbins, scan_step, jnp.int32(0))

        def scat_step(t, _):
            w = payload_ref[pl.ds(t * LANES, LANES)]
            if digit_ref is payload_ref:
                d = (w >> shift) & mask
            else:
                d = (digit_ref[pl.ds(t * LANES, LANES)] >> shift) & mask
            bins = (d << 4) | iota
            r = plsc.load_gather(hist_v, [bins])
            plsc.store_scatter(hist_v, [bins], r + 1)
            if repack:
                out_w = ((w >> 21) << 13) | (w & 0x1FFF)
            elif final:
                out_w = w & 0x1FFF
            else:
                out_w = w
            pos = r if final else transpose_pos(r)
            plsc.store_scatter(out_ref, [pos], out_w)
            return 0
        lax.fori_loop(0, STEPS, scat_step, 0)

    def row_body(rr, _):
        row = wid * ROWS_PER_W + rr
        pltpu.sync_copy(u_hbm.at[row], u_v)
        pltpu.sync_copy(gv_hbm.at[row], gvrow_v)

        def init_step(t, _):
            uu = u_v[pl.ds(t * LANES, LANES)]
            mm = (uu * 8388608.0).astype(jnp.int32)
            q = iota + t * LANES
            pos = transpose_pos(q)
            plsc.store_scatter(a_v, [pos], mm)
            plsc.store_scatter(b_v, [pos], ((mm >> 8) << 13) | q)
            return 0
        lax.fori_loop(0, STEPS, init_step, 0)

        do_pass(a_v, b_v, c_v, 0, 0xFF, 256, repack=False, final=False)
        do_pass(c_v, c_v, a_v, 13, 0xFF, 256, repack=True, final=False)
        do_pass(a_v, a_v, b_v, 13, 0x7F, 128, repack=False, final=True)

        def gath_step(t, _):
            sidx = b_v[pl.ds(t * LANES, LANES)]
            oidx_v[pl.ds(t * LANES, LANES)] = plsc.load_gather(gid_v, [sidx])
            oval_v[pl.ds(t * LANES, LANES)] = plsc.load_gather(gvrow_v, [sidx])
            return 0
        lax.fori_loop(0, CP // LANES, gath_step, 0)

        pltpu.sync_copy(oidx_v, oid_hbm.at[row])
        pltpu.sync_copy(oval_v, oval_hbm.at[row])
        return 0

    lax.fori_loop(0, ROWS_PER_W, row_body, 0)


_sc_sort_gather = functools.partial(
    pl.kernel,
    out_type=(jax.ShapeDtypeStruct((N, CP), jnp.int32),
              jax.ShapeDtypeStruct((N, CP), jnp.float32)),
    mesh=plsc.VectorSubcoreMesh(core_axis_name="c", subcore_axis_name="s"),
    compiler_params=pltpu.CompilerParams(needs_layout_passes=False),
    scratch_types=[
        pltpu.VMEM((G,), jnp.float32),
        pltpu.VMEM((G,), jnp.int32),
        pltpu.VMEM((G,), jnp.int32),
        pltpu.VMEM((G,), jnp.int32),
        pltpu.VMEM((G,), jnp.float32),
        pltpu.VMEM((G,), jnp.int32),
        pltpu.VMEM((256 * LANES,), jnp.int32),
        pltpu.VMEM((CP,), jnp.int32),
        pltpu.VMEM((CP,), jnp.float32),
        pltpu.SemaphoreType.DMA,
    ],
)(_sc_sort_gather_body)


def _tc_tail_body(val_ref, p_ref, b_ref, tot_ref, ct_ref, ti_ref,
                  ch0_ref, ch1_ref, ch2_ref, lab_ref, pm_ref,
                  mtok_ref, mlab_ref):
    blk = val_ref.shape
    col = lax.broadcasted_iota(jnp.int32, blk, 1)
    qmask = col >= PREFIX_LEN

    tot = jnp.maximum(tot_ref[...], 1).astype(jnp.float32)
    down = jnp.minimum(tot, MAX_UMIS)
    p = jnp.minimum(p_ref[...] / 0.5, 1.0)
    down2 = MIN_UMIS + (down - MIN_UMIS) * p
    gene_p = down2 / tot
    gv = jnp.floor(val_ref[...] * gene_p + b_ref[...])
    tr = jnp.round(down2)

    ch0_ref[...] = jnp.log1p(gv) * (~qmask).astype(jnp.float32)
    ch1_ref[...] = qmask.astype(jnp.float32)
    ch2_ref[...] = jnp.log1p(tr)
    lab_ref[...] = jnp.clip(gv, 0.0, GENE_VALUE_VOCAB - 1).astype(jnp.int32)

    ct = ct_ref[...]
    ti = ti_ref[...]
    pm_ref[...] = ((col < PREFIX_LEN) |
                   ((col == C) & (ct >= 0))).astype(jnp.int32)

    mcol = lax.broadcasted_iota(jnp.int32, mtok_ref.shape, 1)
    ct_clip = jnp.maximum(ct, 0)
    ti_clip = jnp.maximum(ti, 0)
    ti_tok = jnp.where(ti >= 0, TISSUE_VOCAB, ti_clip)
    mtok_ref[...] = jnp.where(mcol == 0, ct_clip, ti_tok)
    mlab_ref[...] = jnp.where(mcol == 0, ct_clip, ti_clip)


def _tc_tail(val_p, p_p, b_p, tot, ct, ti):
    BR = 128
    grid = (N // BR,)
    row_spec = pl.BlockSpec((BR, CP), lambda i: (i, 0))
    one_spec = pl.BlockSpec((BR, 1), lambda i: (i, 0))
    two_spec = pl.BlockSpec((BR, M), lambda i: (i, 0))
    return pl.pallas_call(
        _tc_tail_body,
        grid=grid,
        in_specs=[row_spec, row_spec, row_spec, one_spec, one_spec, one_spec],
        out_specs=[row_spec, row_spec, row_spec, row_spec, row_spec,
                   two_spec, two_spec],
        out_shape=[
            jax.ShapeDtypeStruct((N, CP), jnp.float32),
            jax.ShapeDtypeStruct((N, CP), jnp.float32),
            jax.ShapeDtypeStruct((N, CP), jnp.float32),
            jax.ShapeDtypeStruct((N, CP), jnp.int32),
            jax.ShapeDtypeStruct((N, CP), jnp.int32),
            jax.ShapeDtypeStruct((N, M), jnp.int32),
            jax.ShapeDtypeStruct((N, M), jnp.int32),
        ],
        compiler_params=pltpu.CompilerParams(
            dimension_semantics=("parallel",)),
    )(val_p, p_p, b_p, tot, ct, ti)


def kernel(gene_value_ng, total_mrna_umis_n, gene_id_g, cell_type_n, tissue_n):
    n, g = gene_value_ng.shape
    key = jax.random.key(1)
    ks, kp, kb = jax.random.split(key, 3)
    u_ng = jax.random.uniform(ks, (n, g), dtype=jnp.float32)
    p_nc = jax.random.uniform(kp, (n, C))
    b_nc = jax.random.uniform(kb, (n, C))

    gid32 = gene_id_g.astype(jnp.int32)
    oid, oval = _sc_sort_gather(u_ng, gene_value_ng, gid32)

    pad = ((0, 0), (0, CP - C))
    p_p = jnp.pad(p_nc, pad)
    b_p = jnp.pad(b_nc, pad)
    tot = total_mrna_umis_n.astype(jnp.int32).reshape(n, 1)
    ct = cell_type_n.astype(jnp.int32).reshape(n, 1)
    ti = tissue_n.astype(jnp.int32).reshape(n, 1)

    ch0, ch1, ch2, lab, pm, mtok, mlab = _tc_tail(oval, p_p, b_p, tot, ct, ti)

    gene_value_nc3 = jnp.stack([ch0[:, :C], ch1[:, :C], ch2[:, :C]], axis=2)
    gene_id_nc = oid[:, :C]
    gene_label_nc = lab[:, :C]
    prompt_mask_nc = pm.astype(bool)
    return (gene_value_nc3, gene_id_nc, gene_label_nc, mtok, mlab,
            prompt_mask_nc)


# speedup vs baseline: 2.6636x; 1.5854x over previous
import functools

import jax
import jax.numpy as jnp
from jax import lax
from jax.experimental import pallas as pl
from jax.experimental.pallas import tpu as pltpu
from jax.experimental.pallas import tpu_sc as plsc

N = 1024
G = 8192
CONTEXT_LEN = 2048
M = 2
C = CONTEXT_LEN - M
CP = 2048
PREFIX_LEN = 1024
MIN_UMIS = 1000.0
MAX_UMIS = 100000.0
GENE_VALUE_VOCAB = 2048
CELL_TYPE_VOCAB = 100
TISSUE_VOCAB = 50

LANES = 16
STEPS = G // LANES
NW = 32
ROWS_PER_W = N // NW


PAIR = 2
UNROLL = 4


def _sc_sort_gather_body(u_hbm, gv_hbm, gid_hbm, oid_hbm, oval_hbm,
                         u0_v, u1_v, a0_v, a1_v, b0_v, b1_v, c0_v, c1_v,
                         gv0_v, gv1_v, gid_v, h0_v, h1_v,
                         oi0_v, oi1_v, ov0_v, ov1_v, sem):
    wid = lax.axis_index("s") * 2 + lax.axis_index("c")
    iota = lax.iota(jnp.int32, LANES)
    ones = iota - iota + 1

    u_vs = [u0_v, u1_v]
    a_vs = [a0_v, a1_v]
    b_vs = [b0_v, b1_v]
    c_vs = [c0_v, c1_v]
    gv_vs = [gv0_v, gv1_v]
    h_vs = [h0_v, h1_v]
    oi_vs = [oi0_v, oi1_v]
    ov_vs = [ov0_v, ov1_v]

    pltpu.sync_copy(gid_hbm, gid_v)

    def transpose_pos(q):
        return ((q & (STEPS - 1)) << 4) | (q >> 9)

    def do_pass(digit_refs, payload_refs, out_refs, shift, mask, nbins,
                repack, final):
        @plsc.parallel_loop(0, nbins, unroll=UNROLL)
        def zero_step(bb):
            for h in h_vs:
                h[pl.ds(bb * LANES, LANES)] = iota - iota

        @plsc.parallel_loop(0, STEPS, unroll=UNROLL)
        def hist_step(t):
            for j in range(PAIR):
                w = digit_refs[j][pl.ds(t * LANES, LANES)]
                d = (w >> shift) & mask
                plsc.addupdate_scatter(h_vs[j], [(d << 4) | iota], ones)

        @plsc.parallel_loop(0, nbins, unroll=2,
                            carry=(jnp.int32(0), jnp.int32(0)))
        def scan_step(bb, carry):
            out = []
            for j in range(PAIR):
                h = h_vs[j][pl.ds(bb * LANES, LANES)]
                inc = plsc.cumsum(h)
                h_vs[j][pl.ds(bb * LANES, LANES)] = inc - h + carry[j]
                out.append(carry[j] + jnp.sum(h))
            return tuple(out)

        def scat_one(j, t):
            w = payload_refs[j][pl.ds(t * LANES, LANES)]
            if digit_refs[j] is payload_refs[j]:
                d = (w >> shift) & mask
            else:
                d = (digit_refs[j][pl.ds(t * LANES, LANES)] >> shift) & mask
            bins = (d << 4) | iota
            r = plsc.load_gather(h_vs[j], [bins])
            plsc.store_scatter(h_vs[j], [bins], r + 1)
            if repack:
                out_w = ((w >> 21) << 13) | (w & 0x1FFF)
            elif final:
                out_w = w & 0x1FFF
            else:
                out_w = w
            pos = r if final else transpose_pos(r)
            plsc.store_scatter(out_refs[j], [pos], out_w)

        def scat_step(t, _):
            for tt in range(2):
                for j in range(PAIR):
                    scat_one(j, t * 2 + tt)
            return 0
        lax.fori_loop(0, STEPS // 2, scat_step, 0)

    def pair_body(pp, _):
        r0 = wid * ROWS_PER_W + pp * PAIR
        for j in range(PAIR):
            pltpu.sync_copy(u_hbm.at[r0 + j], u_vs[j])
            pltpu.sync_copy(gv_hbm.at[r0 + j], gv_vs[j])

        @plsc.parallel_loop(0, STEPS, unroll=UNROLL)
        def init_step(t):
            q = iota + t * LANES
            pos = transpose_pos(q)
            for j in range(PAIR):
                uu = u_vs[j][pl.ds(t * LANES, LANES)]
                mm = (uu * 8388608.0).astype(jnp.int32)
                plsc.store_scatter(a_vs[j], [pos], mm)
                plsc.store_scatter(b_vs[j], [pos], ((mm >> 8) << 13) | q)

        do_pass(a_vs, b_vs, c_vs, 0, 0xFF, 256, repack=False, final=False)
        do_pass(c_vs, c_vs, a_vs, 13, 0xFF, 256, repack=True, final=False)
        do_pass(a_vs, a_vs, b_vs, 13, 0x7F, 128, repack=False, final=True)

        @plsc.parallel_loop(0, CP // LANES, unroll=UNROLL)
        def gath_step(t):
            sl = pl.ds(t * LANES, LANES)
            for j in range(PAIR):
                sidx = b_vs[j][sl]
                oi_vs[j][sl] = plsc.load_gather(gid_v, [sidx])
                ov_vs[j][sl] = plsc.load_gather(gv_vs[j], [sidx])

        for j in range(PAIR):
            pltpu.sync_copy(oi_vs[j], oid_hbm.at[r0 + j])
            pltpu.sync_copy(ov_vs[j], oval_hbm.at[r0 + j])
        return 0

    lax.fori_loop(0, ROWS_PER_W // PAIR, pair_body, 0)


_sc_sort_gather = functools.partial(
    pl.kernel,
    out_type=(jax.ShapeDtypeStruct((N, CP), jnp.int32),
              jax.ShapeDtypeStruct((N, CP), jnp.float32)),
    mesh=plsc.VectorSubcoreMesh(core_axis_name="c", subcore_axis_name="s"),
    compiler_params=pltpu.CompilerParams(needs_layout_passes=False),
    scratch_types=(
        [pltpu.VMEM((G,), jnp.float32) for _ in range(2)] +
        [pltpu.VMEM((G,), jnp.int32) for _ in range(6)] +
        [pltpu.VMEM((G,), jnp.float32) for _ in range(2)] +
        [pltpu.VMEM((G,), jnp.int32)] +
        [pltpu.VMEM((256 * LANES,), jnp.int32) for _ in range(2)] +
        [pltpu.VMEM((CP,), jnp.int32) for _ in range(2)] +
        [pltpu.VMEM((CP,), jnp.float32) for _ in range(2)] +
        [pltpu.SemaphoreType.DMA]
    ),
)(_sc_sort_gather_body)


def _tc_tail_body(val_ref, p_ref, b_ref, tot_ref, ct_ref, ti_ref,
                  ch0_ref, ch1_ref, ch2_ref, lab_ref, pm_ref,
                  mtok_ref, mlab_ref):
    blk = val_ref.shape
    col = lax.broadcasted_iota(jnp.int32, blk, 1)
    qmask = col >= PREFIX_LEN

    tot = jnp.maximum(tot_ref[...], 1).astype(jnp.float32)
    down = jnp.minimum(tot, MAX_UMIS)
    p = jnp.minimum(p_ref[...] / 0.5, 1.0)
    down2 = MIN_UMIS + (down - MIN_UMIS) * p
    gene_p = down2 / tot
    gv = jnp.floor(val_ref[...] * gene_p + b_ref[...])
    tr = jnp.round(down2)

    ch0_ref[...] = jnp.log1p(gv) * (~qmask).astype(jnp.float32)
    ch1_ref[...] = qmask.astype(jnp.float32)
    ch2_ref[...] = jnp.log1p(tr)
    lab_ref[...] = jnp.clip(gv, 0.0, GENE_VALUE_VOCAB - 1).astype(jnp.int32)

    ct = ct_ref[...]
    ti = ti_ref[...]
    pm_ref[...] = ((col < PREFIX_LEN) |
                   ((col == C) & (ct >= 0))).astype(jnp.int32)

    mcol = lax.broadcasted_iota(jnp.int32, mtok_ref.shape, 1)
    ct_clip = jnp.maximum(ct, 0)
    ti_clip = jnp.maximum(ti, 0)
    ti_tok = jnp.where(ti >= 0, TISSUE_VOCAB, ti_clip)
    mtok_ref[...] = jnp.where(mcol == 0, ct_clip, ti_tok)
    mlab_ref[...] = jnp.where(mcol == 0, ct_clip, ti_clip)


def _tc_tail(val_p, p_p, b_p, tot, ct, ti):
    BR = 128
    grid = (N // BR,)
    row_spec = pl.BlockSpec((BR, CP), lambda i: (i, 0))
    one_spec = pl.BlockSpec((BR, 1), lambda i: (i, 0))
    two_spec = pl.BlockSpec((BR, M), lambda i: (i, 0))
    return pl.pallas_call(
        _tc_tail_body,
        grid=grid,
        in_specs=[row_spec, row_spec, row_spec, one_spec, one_spec, one_spec],
        out_specs=[row_spec, row_spec, row_spec, row_spec, row_spec,
                   two_spec, two_spec],
        out_shape=[
            jax.ShapeDtypeStruct((N, CP), jnp.float32),
            jax.ShapeDtypeStruct((N, CP), jnp.float32),
            jax.ShapeDtypeStruct((N, CP), jnp.float32),
            jax.ShapeDtypeStruct((N, CP), jnp.int32),
            jax.ShapeDtypeStruct((N, CP), jnp.int32),
            jax.ShapeDtypeStruct((N, M), jnp.int32),
            jax.ShapeDtypeStruct((N, M), jnp.int32),
        ],
        compiler_params=pltpu.CompilerParams(
            dimension_semantics=("parallel",)),
    )(val_p, p_p, b_p, tot, ct, ti)


def kernel(gene_value_ng, total_mrna_umis_n, gene_id_g, cell_type_n, tissue_n):
    n, g = gene_value_ng.shape
    key = jax.random.key(1)
    ks, kp, kb = jax.random.split(key, 3)
    u_ng = jax.random.uniform(ks, (n, g), dtype=jnp.float32)
    p_nc = jax.random.uniform(kp, (n, C))
    b_nc = jax.random.uniform(kb, (n, C))

    gid32 = gene_id_g.astype(jnp.int32)
    oid, oval = _sc_sort_gather(u_ng, gene_value_ng, gid32)

    pad = ((0, 0), (0, CP - C))
    p_p = jnp.pad(p_nc, pad)
    b_p = jnp.pad(b_nc, pad)
    tot = total_mrna_umis_n.astype(jnp.int32).reshape(n, 1)
    ct = cell_type_n.astype(jnp.int32).reshape(n, 1)
    ti = tissue_n.astype(jnp.int32).reshape(n, 1)

    ch0, ch1, ch2, lab, pm, mtok, mlab = _tc_tail(oval, p_p, b_p, tot, ct, ti)

    gene_value_nc3 = jnp.stack([ch0[:, :C], ch1[:, :C], ch2[:, :C]], axis=2)
    gene_id_nc = oid[:, :C]
    gene_label_nc = lab[:, :C]
    prompt_mask_nc = pm.astype(bool)
    return (gene_value_nc3, gene_id_nc, gene_label_nc, mtok, mlab,
            prompt_mask_nc)


# speedup vs baseline: 2.8138x; 1.0564x over previous
import functools

import jax
import jax.numpy as jnp
from jax import lax
from jax.experimental import pallas as pl
from jax.experimental.pallas import tpu as pltpu
from jax.experimental.pallas import tpu_sc as plsc

N = 1024
G = 8192
CONTEXT_LEN = 2048
M = 2
C = CONTEXT_LEN - M
CP = 2048
PREFIX_LEN = 1024
MIN_UMIS = 1000.0
MAX_UMIS = 100000.0
GENE_VALUE_VOCAB = 2048
CELL_TYPE_VOCAB = 100
TISSUE_VOCAB = 50

LANES = 16
STEPS = G // LANES
NW = 32
ROWS_PER_W = N // NW

QUAD = 4
UNROLL = 4


def _sc_sort_gather_body(m_hbm, gv_hbm, gid_hbm, oid_hbm, ovb_hbm,
                         x0_v, x1_v, x2_v, x3_v, a0_v, a1_v, a2_v, a3_v,
                         gv0_v, gv1_v, gv2_v, gv3_v, gid_v,
                         h0_v, h1_v, h2_v, h3_v, sem):
    wid = lax.axis_index("s") * 2 + lax.axis_index("c")
    iota = lax.iota(jnp.int32, LANES)
    ones = iota - iota + 1

    x_vs = [x0_v, x1_v, x2_v, x3_v]
    a_vs = [a0_v, a1_v, a2_v, a3_v]
    gv_vs = [gv0_v, gv1_v, gv2_v, gv3_v]
    h_vs = [h0_v, h1_v, h2_v, h3_v]

    pltpu.sync_copy(gid_hbm, gid_v)

    def transpose_pos(q):
        return ((q & (STEPS - 1)) << 4) | (q >> 9)

    def do_pass(in_refs, out_refs, mode):
        nbins = 128 if mode == 3 else 256

        def digit(w):
            if mode == 1:
                return w & 0xFF
            if mode == 2:
                return (w >> 13) & 0xFF
            return (w >> 13) & 0x7F

        @plsc.parallel_loop(0, nbins, unroll=UNROLL)
        def zero_step(bb):
            for h in h_vs:
                h[pl.ds(bb * LANES, LANES)] = iota - iota

        @plsc.parallel_loop(0, STEPS, unroll=UNROLL)
        def hist_step(t):
            for j in range(QUAD):
                d = digit(in_refs[j][pl.ds(t * LANES, LANES)])
                plsc.addupdate_scatter(h_vs[j], [(d << 4) | iota], ones)

        @plsc.parallel_loop(0, nbins, unroll=2,
                            carry=tuple(jnp.int32(0) for _ in range(QUAD)))
        def scan_step(bb, carry):
            out = []
            for j in range(QUAD):
                h = h_vs[j][pl.ds(bb * LANES, LANES)]
                inc = plsc.cumsum(h)
                h_vs[j][pl.ds(bb * LANES, LANES)] = inc - h + carry[j]
                out.append(carry[j] + jnp.sum(h))
            return tuple(out)

        def scat_step(t, _):
            for j in range(QUAD):
                w = in_refs[j][pl.ds(t * LANES, LANES)]
                bins = (digit(w) << 4) | iota
                r = plsc.load_gather(h_vs[j], [bins])
                plsc.store_scatter(h_vs[j], [bins], r + 1)
                if mode == 1:
                    out_w = ((w >> 8) << 13) | (iota * STEPS + t)
                elif mode == 2:
                    out_w = ((w >> 21) << 13) | (w & 0x1FFF)
                else:
                    out_w = w & 0x1FFF
                pos = r if mode == 3 else transpose_pos(r)
                plsc.store_scatter(out_refs[j], [pos], out_w)
            return 0
        lax.fori_loop(0, STEPS, scat_step, 0)

    def quad_body(pp, _):
        r0 = wid * ROWS_PER_W + pp * QUAD
        for j in range(QUAD):
            pltpu.sync_copy(m_hbm.at[r0 + j], x_vs[j])
            pltpu.sync_copy(gv_hbm.at[r0 + j], gv_vs[j])

        @plsc.parallel_loop(0, STEPS, unroll=UNROLL)
        def init_step(t):
            q = iota + t * LANES
            pos = transpose_pos(q)
            for j in range(QUAD):
                plsc.store_scatter(a_vs[j], [pos],
                                   x_vs[j][pl.ds(t * LANES, LANES)])

        do_pass(a_vs, x_vs, 1)
        do_pass(x_vs, a_vs, 2)
        do_pass(a_vs, x_vs, 3)

        @plsc.parallel_loop(0, CP // LANES, unroll=UNROLL)
        def gath_step(t):
            sl = pl.ds(t * LANES, LANES)
            for j in range(QUAD):
                sidx = x_vs[j][sl]
                a_vs[j][sl] = plsc.load_gather(gid_v, [sidx])
                vals = plsc.load_gather(gv_vs[j], [sidx])
                a_vs[j][pl.ds(CP + t * LANES, LANES)] = plsc.bitcast(
                    vals, jnp.int32)

        for j in range(QUAD):
            pltpu.sync_copy(a_vs[j].at[pl.ds(0, CP)], oid_hbm.at[r0 + j])
            pltpu.sync_copy(a_vs[j].at[pl.ds(CP, CP)], ovb_hbm.at[r0 + j])
        return 0

    lax.fori_loop(0, ROWS_PER_W // QUAD, quad_body, 0)


_sc_sort_gather = functools.partial(
    pl.kernel,
    out_type=(jax.ShapeDtypeStruct((N, CP), jnp.int32),
              jax.ShapeDtypeStruct((N, CP), jnp.int32)),
    mesh=plsc.VectorSubcoreMesh(core_axis_name="c", subcore_axis_name="s"),
    compiler_params=pltpu.CompilerParams(needs_layout_passes=False),
    scratch_types=(
        [pltpu.VMEM((G,), jnp.int32) for _ in range(4)] +
        [pltpu.VMEM((G,), jnp.int32) for _ in range(4)] +
        [pltpu.VMEM((G,), jnp.float32) for _ in range(4)] +
        [pltpu.VMEM((G,), jnp.int32)] +
        [pltpu.VMEM((256 * LANES,), jnp.int32) for _ in range(4)] +
        [pltpu.SemaphoreType.DMA]
    ),
)(_sc_sort_gather_body)


def _tc_tail_body(val_ref, p_ref, b_ref, tot_ref, ct_ref, ti_ref,
                  ch0_ref, ch1_ref, ch2_ref, lab_ref, pm_ref,
                  mtok_ref, mlab_ref):
    blk = val_ref.shape
    col = lax.broadcasted_iota(jnp.int32, blk, 1)
    qmask = col >= PREFIX_LEN

    tot = jnp.maximum(tot_ref[...], 1).astype(jnp.float32)
    down = jnp.minimum(tot, MAX_UMIS)
    p = jnp.minimum(p_ref[...] / 0.5, 1.0)
    down2 = MIN_UMIS + (down - MIN_UMIS) * p
    gene_p = down2 / tot
    gv = jnp.floor(val_ref[...] * gene_p + b_ref[...])
    tr = jnp.round(down2)

    ch0_ref[...] = jnp.log1p(gv) * (~qmask).astype(jnp.float32)
    ch1_ref[...] = qmask.astype(jnp.float32)
    ch2_ref[...] = jnp.log1p(tr)
    lab_ref[...] = jnp.clip(gv, 0.0, GENE_VALUE_VOCAB - 1).astype(jnp.int32)

    ct = ct_ref[...]
    ti = ti_ref[...]
    pm_ref[...] = ((col < PREFIX_LEN) |
                   ((col == C) & (ct >= 0))).astype(jnp.int32)

    mcol = lax.broadcasted_iota(jnp.int32, mtok_ref.shape, 1)
    ct_clip = jnp.maximum(ct, 0)
    ti_clip = jnp.maximum(ti, 0)
    ti_tok = jnp.where(ti >= 0, TISSUE_VOCAB, ti_clip)
    mtok_ref[...] = jnp.where(mcol == 0, ct_clip, ti_tok)
    mlab_ref[...] = jnp.where(mcol == 0, ct_clip, ti_clip)


def _tc_tail(val_p, p_p, b_p, tot, ct, ti):
    BR = 128
    grid = (N // BR,)
    row_spec = pl.BlockSpec((BR, CP), lambda i: (i, 0))
    one_spec = pl.BlockSpec((BR, 1), lambda i: (i, 0))
    two_spec = pl.BlockSpec((BR, M), lambda i: (i, 0))
    return pl.pallas_call(
        _tc_tail_body,
        grid=grid,
        in_specs=[row_spec, row_spec, row_spec, one_spec, one_spec, one_spec],
        out_specs=[row_spec, row_spec, row_spec, row_spec, row_spec,
                   two_spec, two_spec],
        out_shape=[
            jax.ShapeDtypeStruct((N, CP), jnp.float32),
            jax.ShapeDtypeStruct((N, CP), jnp.float32),
            jax.ShapeDtypeStruct((N, CP), jnp.float32),
            jax.ShapeDtypeStruct((N, CP), jnp.int32),
            jax.ShapeDtypeStruct((N, CP), jnp.int32),
            jax.ShapeDtypeStruct((N, M), jnp.int32),
            jax.ShapeDtypeStruct((N, M), jnp.int32),
        ],
        compiler_params=pltpu.CompilerParams(
            dimension_semantics=("parallel",)),
    )(val_p, p_p, b_p, tot, ct, ti)


def kernel(gene_value_ng, total_mrna_umis_n, gene_id_g, cell_type_n, tissue_n):
    n, g = gene_value_ng.shape
    key = jax.random.key(1)
    ks, kp, kb = jax.random.split(key, 3)
    u_ng = jax.random.uniform(ks, (n, g), dtype=jnp.float32)
    p_nc = jax.random.uniform(kp, (n, C))
    b_nc = jax.random.uniform(kb, (n, C))

    m_ng = (u_ng * 8388608.0).astype(jnp.int32)
    gid32 = gene_id_g.astype(jnp.int32)
    oid, ovb = _sc_sort_gather(m_ng, gene_value_ng, gid32)
    oval = jax.lax.bitcast_convert_type(ovb, jnp.float32)

    pad = ((0, 0), (0, CP - C))
    p_p = jnp.pad(p_nc, pad)
    b_p = jnp.pad(b_nc, pad)
    tot = total_mrna_umis_n.astype(jnp.int32).reshape(n, 1)
    ct = cell_type_n.astype(jnp.int32).reshape(n, 1)
    ti = tissue_n.astype(jnp.int32).reshape(n, 1)

    ch0, ch1, ch2, lab, pm, mtok, mlab = _tc_tail(oval, p_p, b_p, tot, ct, ti)

    gene_value_nc3 = jnp.stack([ch0[:, :C], ch1[:, :C], ch2[:, :C]], axis=2)
    gene_id_nc = oid[:, :C]
    gene_label_nc = lab[:, :C]
    prompt_mask_nc = pm.astype(bool)
    return (gene_value_nc3, gene_id_nc, gene_label_nc, mtok, mlab,
            prompt_mask_nc)


# speedup vs baseline: 4.4372x; 1.5769x over previous
import functools

import jax
import jax.numpy as jnp
from jax import lax
from jax.experimental import pallas as pl
from jax.experimental.pallas import tpu as pltpu
from jax.experimental.pallas import tpu_sc as plsc

N = 1024
G = 8192
CONTEXT_LEN = 2048
M = 2
C = CONTEXT_LEN - M
CP = 2048
PREFIX_LEN = 1024
MIN_UMIS = 1000.0
MAX_UMIS = 100000.0
GENE_VALUE_VOCAB = 2048
CELL_TYPE_VOCAB = 100
TISSUE_VOCAB = 50

LANES = 16
STEPS = G // LANES
NW = 32
ROWS_PER_W = N // NW

QUAD = 4
UNROLL = 4


def _sc_sort_gather_body(m_hbm, gv_hbm, gid_hbm, oid_hbm, ovb_hbm,
                         x0_v, x1_v, x2_v, x3_v, a0_v, a1_v, a2_v, a3_v,
                         gv0_v, gv1_v, gv2_v, gv3_v, gid_v,
                         h0_v, h1_v, h2_v, h3_v, sem):
    wid = lax.axis_index("s") * 2 + lax.axis_index("c")
    iota = lax.iota(jnp.int32, LANES)
    ones = iota - iota + 1

    x_vs = [x0_v, x1_v, x2_v, x3_v]
    a_vs = [a0_v, a1_v, a2_v, a3_v]
    gv_vs = [gv0_v, gv1_v, gv2_v, gv3_v]
    h_vs = [h0_v, h1_v, h2_v, h3_v]

    pltpu.sync_copy(gid_hbm, gid_v)

    def transpose_pos(q):
        return ((q & (STEPS - 1)) << 4) | (q >> 9)

    def do_pass(in_refs, out_refs, mode):
        nbins = 128 if mode == 3 else 256

        def digit(w):
            if mode == 1:
                return w & 0xFF
            if mode == 2:
                return (w >> 13) & 0xFF
            return (w >> 13) & 0x7F

        @plsc.parallel_loop(0, nbins, unroll=UNROLL)
        def zero_step(bb):
            for h in h_vs:
                h[pl.ds(bb * LANES, LANES)] = iota - iota

        @plsc.parallel_loop(0, STEPS, unroll=UNROLL)
        def hist_step(t):
            for j in range(QUAD):
                d = digit(in_refs[j][pl.ds(t * LANES, LANES)])
                plsc.addupdate_scatter(h_vs[j], [(d << 4) | iota], ones)

        @plsc.parallel_loop(0, nbins, unroll=2,
                            carry=tuple(jnp.int32(0) for _ in range(QUAD)))
        def scan_step(bb, carry):
            out = []
            for j in range(QUAD):
                h = h_vs[j][pl.ds(bb * LANES, LANES)]
                inc = plsc.cumsum(h)
                h_vs[j][pl.ds(bb * LANES, LANES)] = inc - h + carry[j]
                out.append(carry[j] + jnp.sum(h))
            return tuple(out)

        def load_stage(t):
            ws, binss, rs = [], [], []
            for j in range(QUAD):
                w = in_refs[j][pl.ds(t * LANES, LANES)]
                bins = (digit(w) << 4) | iota
                ws.append(w)
                binss.append(bins)
                rs.append(plsc.load_gather(h_vs[j], [bins]))
            return tuple(ws), tuple(binss), tuple(rs)

        def store_stage(t, ws, binss, rs):
            for j in range(QUAD):
                w, bins, r = ws[j], binss[j], rs[j]
                plsc.store_scatter(h_vs[j], [bins], r + 1)
                if mode == 1:
                    out_w = ((w >> 8) << 13) | (iota * STEPS + t)
                elif mode == 2:
                    out_w = ((w >> 21) << 13) | (w & 0x1FFF)
                else:
                    out_w = w & 0x1FFF
                pos = r if mode == 3 else transpose_pos(r)
                plsc.store_scatter(out_refs[j], [pos], out_w)

        def scat_step(t, carry):
            ws, binss, rs = carry
            store_stage(t, ws, binss, rs)
            return load_stage(t + 1)

        last = lax.fori_loop(0, STEPS - 1, scat_step, load_stage(0))
        store_stage(STEPS - 1, *last)

    def quad_body(pp, _):
        r0 = wid * ROWS_PER_W + pp * QUAD
        for j in range(QUAD):
            pltpu.sync_copy(m_hbm.at[r0 + j], x_vs[j])
            pltpu.sync_copy(gv_hbm.at[r0 + j], gv_vs[j])

        @plsc.parallel_loop(0, STEPS, unroll=UNROLL)
        def init_step(t):
            q = iota + t * LANES
            pos = transpose_pos(q)
            for j in range(QUAD):
                plsc.store_scatter(a_vs[j], [pos],
                                   x_vs[j][pl.ds(t * LANES, LANES)])

        do_pass(a_vs, x_vs, 1)
        do_pass(x_vs, a_vs, 2)
        do_pass(a_vs, x_vs, 3)

        @plsc.parallel_loop(0, CP // LANES, unroll=UNROLL)
        def gath_step(t):
            sl = pl.ds(t * LANES, LANES)
            for j in range(QUAD):
                sidx = x_vs[j][sl]
                a_vs[j][sl] = plsc.load_gather(gid_v, [sidx])
                vals = plsc.load_gather(gv_vs[j], [sidx])
                a_vs[j][pl.ds(CP + t * LANES, LANES)] = plsc.bitcast(
                    vals, jnp.int32)

        for j in range(QUAD):
            pltpu.sync_copy(a_vs[j].at[pl.ds(0, CP)], oid_hbm.at[r0 + j])
            pltpu.sync_copy(a_vs[j].at[pl.ds(CP, CP)], ovb_hbm.at[r0 + j])
        return 0

    lax.fori_loop(0, ROWS_PER_W // QUAD, quad_body, 0)


_sc_sort_gather = functools.partial(
    pl.kernel,
    out_type=(jax.ShapeDtypeStruct((N, CP), jnp.int32),
              jax.ShapeDtypeStruct((N, CP), jnp.int32)),
    mesh=plsc.VectorSubcoreMesh(core_axis_name="c", subcore_axis_name="s"),
    compiler_params=pltpu.CompilerParams(needs_layout_passes=False),
    scratch_types=(
        [pltpu.VMEM((G,), jnp.int32) for _ in range(4)] +
        [pltpu.VMEM((G,), jnp.int32) for _ in range(4)] +
        [pltpu.VMEM((G,), jnp.float32) for _ in range(4)] +
        [pltpu.VMEM((G,), jnp.int32)] +
        [pltpu.VMEM((256 * LANES,), jnp.int32) for _ in range(4)] +
        [pltpu.SemaphoreType.DMA]
    ),
)(_sc_sort_gather_body)


def _tc_tail_body(val_ref, p_ref, b_ref, tot_ref, ct_ref, ti_ref,
                  ch0_ref, ch1_ref, ch2_ref, lab_ref, pm_ref,
                  mtok_ref, mlab_ref):
    blk = val_ref.shape
    col = lax.broadcasted_iota(jnp.int32, blk, 1)
    qmask = col >= PREFIX_LEN

    tot = jnp.maximum(tot_ref[...], 1).astype(jnp.float32)
    down = jnp.minimum(tot, MAX_UMIS)
    p = jnp.minimum(p_ref[...] / 0.5, 1.0)
    down2 = MIN_UMIS + (down - MIN_UMIS) * p
    gene_p = down2 / tot
    gv = jnp.floor(val_ref[...] * gene_p + b_ref[...])
    tr = jnp.round(down2)

    ch0_ref[...] = jnp.log1p(gv) * (~qmask).astype(jnp.float32)
    ch1_ref[...] = qmask.astype(jnp.float32)
    ch2_ref[...] = jnp.log1p(tr)
    lab_ref[...] = jnp.clip(gv, 0.0, GENE_VALUE_VOCAB - 1).astype(jnp.int32)

    ct = ct_ref[...]
    ti = ti_ref[...]
    pm_ref[...] = ((col < PREFIX_LEN) |
                   ((col == C) & (ct >= 0))).astype(jnp.int32)

    mcol = lax.broadcasted_iota(jnp.int32, mtok_ref.shape, 1)
    ct_clip = jnp.maximum(ct, 0)
    ti_clip = jnp.maximum(ti, 0)
    ti_tok = jnp.where(ti >= 0, TISSUE_VOCAB, ti_clip)
    mtok_ref[...] = jnp.where(mcol == 0, ct_clip, ti_tok)
    mlab_ref[...] = jnp.where(mcol == 0, ct_clip, ti_clip)


def _tc_tail(val_p, p_p, b_p, tot, ct, ti):
    BR = 128
    grid = (N // BR,)
    row_spec = pl.BlockSpec((BR, CP), lambda i: (i, 0))
    one_spec = pl.BlockSpec((BR, 1), lambda i: (i, 0))
    two_spec = pl.BlockSpec((BR, M), lambda i: (i, 0))
    return pl.pallas_call(
        _tc_tail_body,
        grid=grid,
        in_specs=[row_spec, row_spec, row_spec, one_spec, one_spec, one_spec],
        out_specs=[row_spec, row_spec, row_spec, row_spec, row_spec,
                   two_spec, two_spec],
        out_shape=[
            jax.ShapeDtypeStruct((N, CP), jnp.float32),
            jax.ShapeDtypeStruct((N, CP), jnp.float32),
            jax.ShapeDtypeStruct((N, CP), jnp.float32),
            jax.ShapeDtypeStruct((N, CP), jnp.int32),
            jax.ShapeDtypeStruct((N, CP), jnp.int32),
            jax.ShapeDtypeStruct((N, M), jnp.int32),
            jax.ShapeDtypeStruct((N, M), jnp.int32),
        ],
        compiler_params=pltpu.CompilerParams(
            dimension_semantics=("parallel",)),
    )(val_p, p_p, b_p, tot, ct, ti)


def kernel(gene_value_ng, total_mrna_umis_n, gene_id_g, cell_type_n, tissue_n):
    n, g = gene_value_ng.shape
    key = jax.random.key(1)
    ks, kp, kb = jax.random.split(key, 3)
    u_ng = jax.random.uniform(ks, (n, g), dtype=jnp.float32)
    p_nc = jax.random.uniform(kp, (n, C))
    b_nc = jax.random.uniform(kb, (n, C))

    m_ng = (u_ng * 8388608.0).astype(jnp.int32)
    gid32 = gene_id_g.astype(jnp.int32)
    oid, ovb = _sc_sort_gather(m_ng, gene_value_ng, gid32)
    oval = jax.lax.bitcast_convert_type(ovb, jnp.float32)

    pad = ((0, 0), (0, CP - C))
    p_p = jnp.pad(p_nc, pad)
    b_p = jnp.pad(b_nc, pad)
    tot = total_mrna_umis_n.astype(jnp.int32).reshape(n, 1)
    ct = cell_type_n.astype(jnp.int32).reshape(n, 1)
    ti = tissue_n.astype(jnp.int32).reshape(n, 1)

    ch0, ch1, ch2, lab, pm, mtok, mlab = _tc_tail(oval, p_p, b_p, tot, ct, ti)

    gene_value_nc3 = jnp.stack([ch0[:, :C], ch1[:, :C], ch2[:, :C]], axis=2)
    gene_id_nc = oid[:, :C]
    gene_label_nc = lab[:, :C]
    prompt_mask_nc = pm.astype(bool)
    return (gene_value_nc3, gene_id_nc, gene_label_nc, mtok, mlab,
            prompt_mask_nc)


# speedup vs baseline: 4.7001x; 1.0593x over previous
import functools

import jax
import jax.numpy as jnp
from jax import lax
from jax.experimental import pallas as pl
from jax.experimental.pallas import tpu as pltpu
from jax.experimental.pallas import tpu_sc as plsc

N = 1024
G = 8192
CONTEXT_LEN = 2048
M = 2
C = CONTEXT_LEN - M
CP = 2048
PREFIX_LEN = 1024
MIN_UMIS = 1000.0
MAX_UMIS = 100000.0
GENE_VALUE_VOCAB = 2048
CELL_TYPE_VOCAB = 100
TISSUE_VOCAB = 50

LANES = 16
STEPS = G // LANES
NW = 32
ROWS_PER_W = N // NW

QUAD = 4
UNROLL = 4


def _sc_sort_gather_body(m_hbm, gv_hbm, gid_hbm, oid_hbm, ovb_hbm,
                         x0_v, x1_v, x2_v, x3_v, a0_v, a1_v, a2_v, a3_v,
                         gv0_v, gv1_v, gv2_v, gv3_v, gid_v,
                         h0_v, h1_v, h2_v, h3_v, sem):
    wid = lax.axis_index("s") * 2 + lax.axis_index("c")
    iota = lax.iota(jnp.int32, LANES)
    ones = iota - iota + 1

    x_vs = [x0_v, x1_v, x2_v, x3_v]
    a_vs = [a0_v, a1_v, a2_v, a3_v]
    gv_vs = [gv0_v, gv1_v, gv2_v, gv3_v]
    h_vs = [h0_v, h1_v, h2_v, h3_v]

    pltpu.sync_copy(gid_hbm, gid_v)

    def transpose_pos(q):
        return ((q & (STEPS - 1)) << 4) | (q >> 9)

    def do_pass(in_refs, out_refs, mode):
        nbins = 128 if mode == 3 else 256

        def digit(w):
            if mode == 1:
                return w & 0xFF
            if mode == 2:
                return (w >> 13) & 0xFF
            return (w >> 13) & 0x7F

        @plsc.parallel_loop(0, nbins, unroll=UNROLL)
        def zero_step(bb):
            for h in h_vs:
                h[pl.ds(bb * LANES, LANES)] = iota - iota

        @plsc.parallel_loop(0, STEPS, unroll=UNROLL)
        def hist_step(t):
            for j in range(QUAD):
                d = digit(in_refs[j][pl.ds(t * LANES, LANES)])
                plsc.addupdate_scatter(h_vs[j], [(d << 4) | iota], ones)

        @plsc.parallel_loop(0, nbins, unroll=2,
                            carry=tuple(jnp.int32(0) for _ in range(QUAD)))
        def scan_step(bb, carry):
            out = []
            for j in range(QUAD):
                h = h_vs[j][pl.ds(bb * LANES, LANES)]
                inc = plsc.cumsum(h)
                h_vs[j][pl.ds(bb * LANES, LANES)] = inc - h + carry[j]
                out.append(carry[j] + jnp.sum(h))
            return tuple(out)

        def load_stage(t):
            ws, binss, rs = [], [], []
            for j in range(QUAD):
                w = in_refs[j][pl.ds(t * LANES, LANES)]
                bins = (digit(w) << 4) | iota
                ws.append(w)
                binss.append(bins)
                rs.append(plsc.load_gather(h_vs[j], [bins]))
            return tuple(ws), tuple(binss), tuple(rs)

        def store_stage(t, ws, binss, rs):
            for j in range(QUAD):
                w, bins, r = ws[j], binss[j], rs[j]
                plsc.store_scatter(h_vs[j], [bins], r + 1)
                if mode == 1:
                    out_w = ((w >> 8) << 13) | (iota * STEPS + t)
                elif mode == 2:
                    out_w = ((w >> 21) << 13) | (w & 0x1FFF)
                else:
                    out_w = w & 0x1FFF
                pos = r if mode == 3 else transpose_pos(r)
                plsc.store_scatter(out_refs[j], [pos], out_w)

        def scat_step(k, carry):
            store_stage(2 * k, *carry)
            mid = load_stage(2 * k + 1)
            store_stage(2 * k + 1, *mid)
            return load_stage(2 * k + 2)

        last = lax.fori_loop(0, STEPS // 2 - 1, scat_step, load_stage(0))
        store_stage(STEPS - 2, *last)
        store_stage(STEPS - 1, *load_stage(STEPS - 1))

    def quad_body(pp, _):
        r0 = wid * ROWS_PER_W + pp * QUAD
        gv_dmas = [pltpu.async_copy(gv_hbm.at[r0 + j], gv_vs[j], sem)
                   for j in range(QUAD)]
        for j in range(QUAD):
            pltpu.sync_copy(m_hbm.at[r0 + j], x_vs[j])

        @plsc.parallel_loop(0, STEPS, unroll=UNROLL)
        def init_step(t):
            q = iota + t * LANES
            pos = transpose_pos(q)
            for j in range(QUAD):
                plsc.store_scatter(a_vs[j], [pos],
                                   x_vs[j][pl.ds(t * LANES, LANES)])

        do_pass(a_vs, x_vs, 1)
        do_pass(x_vs, a_vs, 2)
        do_pass(a_vs, x_vs, 3)

        for d in gv_dmas:
            d.wait()

        @plsc.parallel_loop(0, CP // LANES, unroll=UNROLL)
        def gath_step(t):
            sl = pl.ds(t * LANES, LANES)
            for j in range(QUAD):
                sidx = x_vs[j][sl]
                a_vs[j][sl] = plsc.load_gather(gid_v, [sidx])
                vals = plsc.load_gather(gv_vs[j], [sidx])
                a_vs[j][pl.ds(CP + t * LANES, LANES)] = plsc.bitcast(
                    vals, jnp.int32)

        for j in range(QUAD):
            pltpu.sync_copy(a_vs[j].at[pl.ds(0, CP)], oid_hbm.at[r0 + j])
            pltpu.sync_copy(a_vs[j].at[pl.ds(CP, CP)], ovb_hbm.at[r0 + j])
        return 0

    lax.fori_loop(0, ROWS_PER_W // QUAD, quad_body, 0)


_sc_sort_gather = functools.partial(
    pl.kernel,
    out_type=(jax.ShapeDtypeStruct((N, CP), jnp.int32),
              jax.ShapeDtypeStruct((N, CP), jnp.int32)),
    mesh=plsc.VectorSubcoreMesh(core_axis_name="c", subcore_axis_name="s"),
    compiler_params=pltpu.CompilerParams(needs_layout_passes=False),
    scratch_types=(
        [pltpu.VMEM((G,), jnp.int32) for _ in range(4)] +
        [pltpu.VMEM((G,), jnp.int32) for _ in range(4)] +
        [pltpu.VMEM((G,), jnp.float32) for _ in range(4)] +
        [pltpu.VMEM((G,), jnp.int32)] +
        [pltpu.VMEM((256 * LANES,), jnp.int32) for _ in range(4)] +
        [pltpu.SemaphoreType.DMA]
    ),
)(_sc_sort_gather_body)


def _tc_tail_body(val_ref, p_ref, b_ref, tot_ref, ct_ref, ti_ref,
                  ch0_ref, ch1_ref, ch2_ref, lab_ref, pm_ref,
                  mtok_ref, mlab_ref):
    blk = val_ref.shape
    col = lax.broadcasted_iota(jnp.int32, blk, 1)
    qmask = col >= PREFIX_LEN

    tot = jnp.maximum(tot_ref[...], 1).astype(jnp.float32)
    down = jnp.minimum(tot, MAX_UMIS)
    p = jnp.minimum(p_ref[...] / 0.5, 1.0)
    down2 = MIN_UMIS + (down - MIN_UMIS) * p
    gene_p = down2 / tot
    gv = jnp.floor(val_ref[...] * gene_p + b_ref[...])
    tr = jnp.round(down2)

    ch0_ref[...] = jnp.log1p(gv) * (~qmask).astype(jnp.float32)
    ch1_ref[...] = qmask.astype(jnp.float32)
    ch2_ref[...] = jnp.log1p(tr)
    lab_ref[...] = jnp.clip(gv, 0.0, GENE_VALUE_VOCAB - 1).astype(jnp.int32)

    ct = ct_ref[...]
    ti = ti_ref[...]
    pm_ref[...] = ((col < PREFIX_LEN) |
                   ((col == C) & (ct >= 0))).astype(jnp.int32)

    mcol = lax.broadcasted_iota(jnp.int32, mtok_ref.shape, 1)
    ct_clip = jnp.maximum(ct, 0)
    ti_clip = jnp.maximum(ti, 0)
    ti_tok = jnp.where(ti >= 0, TISSUE_VOCAB, ti_clip)
    mtok_ref[...] = jnp.where(mcol == 0, ct_clip, ti_tok)
    mlab_ref[...] = jnp.where(mcol == 0, ct_clip, ti_clip)


def _tc_tail(val_p, p_p, b_p, tot, ct, ti):
    BR = 128
    grid = (N // BR,)
    row_spec = pl.BlockSpec((BR, CP), lambda i: (i, 0))
    one_spec = pl.BlockSpec((BR, 1), lambda i: (i, 0))
    two_spec = pl.BlockSpec((BR, M), lambda i: (i, 0))
    return pl.pallas_call(
        _tc_tail_body,
        grid=grid,
        in_specs=[row_spec, row_spec, row_spec, one_spec, one_spec, one_spec],
        out_specs=[row_spec, row_spec, row_spec, row_spec, row_spec,
                   two_spec, two_spec],
        out_shape=[
            jax.ShapeDtypeStruct((N, CP), jnp.float32),
            jax.ShapeDtypeStruct((N, CP), jnp.float32),
            jax.ShapeDtypeStruct((N, CP), jnp.float32),
            jax.ShapeDtypeStruct((N, CP), jnp.int32),
            jax.ShapeDtypeStruct((N, CP), jnp.int32),
            jax.ShapeDtypeStruct((N, M), jnp.int32),
            jax.ShapeDtypeStruct((N, M), jnp.int32),
        ],
        compiler_params=pltpu.CompilerParams(
            dimension_semantics=("parallel",)),
    )(val_p, p_p, b_p, tot, ct, ti)


def kernel(gene_value_ng, total_mrna_umis_n, gene_id_g, cell_type_n, tissue_n):
    n, g = gene_value_ng.shape
    key = jax.random.key(1)
    ks, kp, kb = jax.random.split(key, 3)
    u_ng = jax.random.uniform(ks, (n, g), dtype=jnp.float32)
    p_nc = jax.random.uniform(kp, (n, C))
    b_nc = jax.random.uniform(kb, (n, C))

    m_ng = (u_ng * 8388608.0).astype(jnp.int32)
    gid32 = gene_id_g.astype(jnp.int32)
    oid, ovb = _sc_sort_gather(m_ng, gene_value_ng, gid32)
    oval = jax.lax.bitcast_convert_type(ovb, jnp.float32)

    pad = ((0, 0), (0, CP - C))
    p_p = jnp.pad(p_nc, pad)
    b_p = jnp.pad(b_nc, pad)
    tot = total_mrna_umis_n.astype(jnp.int32).reshape(n, 1)
    ct = cell_type_n.astype(jnp.int32).reshape(n, 1)
    ti = tissue_n.astype(jnp.int32).reshape(n, 1)

    ch0, ch1, ch2, lab, pm, mtok, mlab = _tc_tail(oval, p_p, b_p, tot, ct, ti)

    gene_value_nc3 = jnp.stack([ch0[:, :C], ch1[:, :C], ch2[:, :C]], axis=2)
    gene_id_nc = oid[:, :C]
    gene_label_nc = lab[:, :C]
    prompt_mask_nc = pm.astype(bool)
    return (gene_value_nc3, gene_id_nc, gene_label_nc, mtok, mlab,
            prompt_mask_nc)
